# Initial kernel scaffold; baseline (speedup 1.0000x reference)
import jax
import jax.numpy as jnp
from jax.experimental import pallas as pl


def kernel(x, w1, b1, Wa, ba, Wb, bb, Wc, bc):
    raise NotImplementedError("write your pallas kernel here")


# speedup vs baseline: 1.3008x; 1.3008x over previous
import functools

import jax
import jax.numpy as jnp
from jax.experimental import pallas as pl
from jax.experimental.pallas import tpu as pltpu

B, N, L, R, C = 16, 8192, 512, 5, 2
N_BLK = 2048
NB = N // N_BLK


def _chowder_kernel(x_ref, w1_ref, b1_ref, Wa_ref, ba_ref, Wb_ref, bb_ref,
                    Wc_ref, bc_ref, out_ref, s_scr):
    nb = pl.program_id(1)
    xblk = x_ref[0]
    w = w1_ref[:].reshape(L, 1)
    s = jnp.dot(xblk, w, preferred_element_type=jnp.float32)
    s_scr[nb, :] = s[:, 0] + b1_ref[0]

    @pl.when(nb == NB - 1)
    def _finish():
        vals = s_scr[:, :]
        gidx = jax.lax.broadcasted_iota(jnp.int32, (NB, N_BLK), 0) * N_BLK \
            + jax.lax.broadcasted_iota(jnp.int32, (NB, N_BLK), 1)
        big = jnp.int32(2**30)

        def take_extreme(v, sign):
            m = jnp.max(v) if sign > 0 else jnp.min(v)
            fi = jnp.min(jnp.where(v == m, gidx, big))
            v2 = jnp.where(gidx == fi,
                           jnp.float32(-jnp.inf) if sign > 0 else jnp.float32(jnp.inf),
                           v)
            return m, v2

        maxs = []
        v = vals
        for _ in range(R):
            m, v = take_extreme(v, +1)
            maxs.append(m)
        mins = []
        v = vals
        for _ in range(R):
            m, v = take_extreme(v, -1)
            mins.append(m)

        cat = jnp.stack(mins + maxs).reshape(1, 2 * R)
        h = jnp.dot(cat, Wa_ref[:].T, preferred_element_type=jnp.float32) + ba_ref[:]
        h = jnp.dot(h, Wb_ref[:].T, preferred_element_type=jnp.float32) + bb_ref[:]
        o = jnp.dot(h, Wc_ref[:].T, preferred_element_type=jnp.float32) + bc_ref[:]
        out_ref[0, 0, :] = o[0]


@functools.partial(jax.jit, static_argnames=())
def _chowder(x, w1, b1, Wa, ba, Wb, bb, Wc, bc):
    out = pl.pallas_call(
        _chowder_kernel,
        grid=(B, NB),
        in_specs=[
            pl.BlockSpec((1, N_BLK, L), lambda b, nb: (b, nb, 0)),
            pl.BlockSpec((L,), lambda b, nb: (0,)),
            pl.BlockSpec((1,), lambda b, nb: (0,)),
            pl.BlockSpec((200, 2 * R), lambda b, nb: (0, 0)),
            pl.BlockSpec((200,), lambda b, nb: (0,)),
            pl.BlockSpec((100, 200), lambda b, nb: (0, 0)),
            pl.BlockSpec((100,), lambda b, nb: (0,)),
            pl.BlockSpec((C, 100), lambda b, nb: (0, 0)),
            pl.BlockSpec((C,), lambda b, nb: (0,)),
        ],
        out_specs=pl.BlockSpec((1, 1, C), lambda b, nb: (b, 0, 0)),
        out_shape=jax.ShapeDtypeStruct((B, 1, C), jnp.float32),
        scratch_shapes=[pltpu.VMEM((NB, N_BLK), jnp.float32)],
        compiler_params=pltpu.CompilerParams(
            dimension_semantics=("arbitrary", "arbitrary"),
        ),
    )(x, w1, b1, Wa, ba, Wb, bb, Wc, bc)
    return out


def kernel(x, w1, b1, Wa, ba, Wb, bb, Wc, bc):
    out = _chowder(x.astype(jnp.float32), w1, b1, Wa, ba, Wb, bb, Wc, bc)
    return (out, None)


# speedup vs baseline: 1.4274x; 1.0973x over previous
import functools

import jax
import jax.numpy as jnp
from jax.experimental import pallas as pl
from jax.experimental.pallas import tpu as pltpu

B, N, L, R, C = 16, 8192, 512, 5, 2
N_BLK = 4096
NB = N // N_BLK


def _chowder_kernel(x_ref, w1_ref, b1_ref, Wa_ref, ba_ref, Wb_ref, bb_ref,
                    Wc_ref, bc_ref, out_ref, s_scr):
    nb = pl.program_id(1)
    xblk = x_ref[0]
    w = w1_ref[:].reshape(L, 1)
    s = jnp.dot(xblk, w, preferred_element_type=jnp.float32)
    s_scr[nb, :] = s[:, 0] + b1_ref[0]

    @pl.when(nb == NB - 1)
    def _finish():
        vals = s_scr[:, :]
        gidx = jax.lax.broadcasted_iota(jnp.int32, (NB, N_BLK), 0) * N_BLK \
            + jax.lax.broadcasted_iota(jnp.int32, (NB, N_BLK), 1)
        big = jnp.int32(2**30)

        def take_extreme(v, sign):
            m = jnp.max(v) if sign > 0 else jnp.min(v)
            fi = jnp.min(jnp.where(v == m, gidx, big))
            v2 = jnp.where(gidx == fi,
                           jnp.float32(-jnp.inf) if sign > 0 else jnp.float32(jnp.inf),
                           v)
            return m, v2

        maxs = []
        v = vals
        for _ in range(R):
            m, v = take_extreme(v, +1)
            maxs.append(m)
        mins = []
        v = vals
        for _ in range(R):
            m, v = take_extreme(v, -1)
            mins.append(m)

        cat = jnp.stack(mins + maxs).reshape(1, 2 * R)
        h = jnp.dot(cat, Wa_ref[:].T, preferred_element_type=jnp.float32) + ba_ref[:]
        h = jnp.dot(h, Wb_ref[:].T, preferred_element_type=jnp.float32) + bb_ref[:]
        o = jnp.dot(h, Wc_ref[:].T, preferred_element_type=jnp.float32) + bc_ref[:]
        out_ref[0, 0, :] = o[0]


@functools.partial(jax.jit, static_argnames=())
def _chowder(x, w1, b1, Wa, ba, Wb, bb, Wc, bc):
    out = pl.pallas_call(
        _chowder_kernel,
        grid=(B, NB),
        in_specs=[
            pl.BlockSpec((1, N_BLK, L), lambda b, nb: (b, nb, 0)),
            pl.BlockSpec((L,), lambda b, nb: (0,)),
            pl.BlockSpec((1,), lambda b, nb: (0,)),
            pl.BlockSpec((200, 2 * R), lambda b, nb: (0, 0)),
            pl.BlockSpec((200,), lambda b, nb: (0,)),
            pl.BlockSpec((100, 200), lambda b, nb: (0, 0)),
            pl.BlockSpec((100,), lambda b, nb: (0,)),
            pl.BlockSpec((C, 100), lambda b, nb: (0, 0)),
            pl.BlockSpec((C,), lambda b, nb: (0,)),
        ],
        out_specs=pl.BlockSpec((1, 1, C), lambda b, nb: (b, 0, 0)),
        out_shape=jax.ShapeDtypeStruct((B, 1, C), jnp.float32),
        scratch_shapes=[pltpu.VMEM((NB, N_BLK), jnp.float32)],
        compiler_params=pltpu.CompilerParams(
            dimension_semantics=("arbitrary", "arbitrary"),
        ),
    )(x, w1, b1, Wa, ba, Wb, bb, Wc, bc)
    return out


def kernel(x, w1, b1, Wa, ba, Wb, bb, Wc, bc):
    out = _chowder(x.astype(jnp.float32), w1, b1, Wa, ba, Wb, bb, Wc, bc)
    return (out, None)


# speedup vs baseline: 1.6172x; 1.1330x over previous
import functools

import jax
import jax.numpy as jnp
from jax.experimental import pallas as pl
from jax.experimental.pallas import tpu as pltpu

B, N, L, R, C = 16, 8192, 512, 5, 2
N_BLK = 8192
NB = N // N_BLK


def _chowder_kernel(x_ref, w1_ref, b1_ref, Wa_ref, ba_ref, Wb_ref, bb_ref,
                    Wc_ref, bc_ref, out_ref, s_scr):
    nb = pl.program_id(1)
    xblk = x_ref[0]
    w = w1_ref[:].reshape(L, 1)
    s = jnp.dot(xblk, w, preferred_element_type=jnp.float32)
    s_scr[nb, :] = s[:, 0] + b1_ref[0]

    @pl.when(nb == NB - 1)
    def _finish():
        vals = s_scr[:, :]
        gidx = jax.lax.broadcasted_iota(jnp.int32, (NB, N_BLK), 0) * N_BLK \
            + jax.lax.broadcasted_iota(jnp.int32, (NB, N_BLK), 1)
        big = jnp.int32(2**30)

        def take_extreme(v, sign):
            m = jnp.max(v) if sign > 0 else jnp.min(v)
            fi = jnp.min(jnp.where(v == m, gidx, big))
            v2 = jnp.where(gidx == fi,
                           jnp.float32(-jnp.inf) if sign > 0 else jnp.float32(jnp.inf),
                           v)
            return m, v2

        maxs = []
        v = vals
        for _ in range(R):
            m, v = take_extreme(v, +1)
            maxs.append(m)
        mins = []
        v = vals
        for _ in range(R):
            m, v = take_extreme(v, -1)
            mins.append(m)

        cat = jnp.stack(mins + maxs).reshape(1, 2 * R)
        h = jnp.dot(cat, Wa_ref[:].T, preferred_element_type=jnp.float32) + ba_ref[:]
        h = jnp.dot(h, Wb_ref[:].T, preferred_element_type=jnp.float32) + bb_ref[:]
        o = jnp.dot(h, Wc_ref[:].T, preferred_element_type=jnp.float32) + bc_ref[:]
        out_ref[0, 0, :] = o[0]


@functools.partial(jax.jit, static_argnames=())
def _chowder(x, w1, b1, Wa, ba, Wb, bb, Wc, bc):
    out = pl.pallas_call(
        _chowder_kernel,
        grid=(B, NB),
        in_specs=[
            pl.BlockSpec((1, N_BLK, L), lambda b, nb: (b, nb, 0)),
            pl.BlockSpec((L,), lambda b, nb: (0,)),
            pl.BlockSpec((1,), lambda b, nb: (0,)),
            pl.BlockSpec((200, 2 * R), lambda b, nb: (0, 0)),
            pl.BlockSpec((200,), lambda b, nb: (0,)),
            pl.BlockSpec((100, 200), lambda b, nb: (0, 0)),
            pl.BlockSpec((100,), lambda b, nb: (0,)),
            pl.BlockSpec((C, 100), lambda b, nb: (0, 0)),
            pl.BlockSpec((C,), lambda b, nb: (0,)),
        ],
        out_specs=pl.BlockSpec((1, 1, C), lambda b, nb: (b, 0, 0)),
        out_shape=jax.ShapeDtypeStruct((B, 1, C), jnp.float32),
        scratch_shapes=[pltpu.VMEM((NB, N_BLK), jnp.float32)],
        compiler_params=pltpu.CompilerParams(
            dimension_semantics=("arbitrary", "arbitrary"),
        ),
    )(x, w1, b1, Wa, ba, Wb, bb, Wc, bc)
    return out


def kernel(x, w1, b1, Wa, ba, Wb, bb, Wc, bc):
    out = _chowder(x.astype(jnp.float32), w1, b1, Wa, ba, Wb, bb, Wc, bc)
    return (out, None)
